# baseline (speedup 1.0000x reference)
import functools

import jax
import jax.numpy as jnp
from jax import lax
from jax.experimental import pallas as pl
from jax.experimental.pallas import tpu as pltpu
from jax.experimental.pallas import tpu_sc as plsc

B = 1024
ENT_DIM = 256
REL_DIM = 128
Q_DIM = 768
HIST_DIM = 768
K_ENT = 100000

KT = 2000
PAIR = 2 * KT
NPAIR = K_ENT // PAIR
GRID = NPAIR + 1

_F32_BIG = 3.0e38


def _mm_bf16(a, b_f32):
    return jax.lax.dot_general(
        a, b_f32.astype(jnp.bfloat16), (((1,), (1,)), ((), ())),
        preferred_element_type=jnp.float32)


def _ksq2(tile):
    s = 0.5 * jnp.sum(tile * tile, axis=1)
    return s[None, :]


def _nn_body(actions_ref, pos_ref, q_ref, table_ref, w_ref, b_ref,
             idx_out_ref, proj_out_ref,
             newpos_ref, qsq2_ref, best_ref, bidx_ref, dotsb_ref, ksqb_ref):
    i = pl.program_id(0)

    @pl.when(i == 0)
    def _init():
        phase = actions_ref[...]
        cos_p = jnp.cos(phase)
        sin_p = jnp.sin(phase)
        pos = pos_ref[...]
        re = pos[:, :REL_DIM]
        im = pos[:, REL_DIM:]
        re2 = re * cos_p - im * sin_p
        im2 = re * sin_p + im * cos_p
        new_pos = jnp.concatenate([re2, im2], axis=1)
        newpos_ref[...] = new_pos.astype(jnp.bfloat16)
        qsq2_ref[...] = 0.5 * jnp.sum(new_pos * new_pos, axis=1, keepdims=True)
        best_ref[...] = jnp.full((B, 1), jnp.inf, jnp.float32)
        bidx_ref[...] = jnp.zeros((B, 1), jnp.int32)
        dotsb_ref[...] = jnp.full((B, PAIR), -_F32_BIG, jnp.float32)
        ksqb_ref[...] = jnp.zeros((8, PAIR), jnp.float32)
        proj = jax.lax.dot_general(
            q_ref[...].astype(jnp.bfloat16), w_ref[:Q_DIM, :].astype(jnp.bfloat16),
            (((1,), (0,)), ((), ())), preferred_element_type=jnp.float32)
        proj += jax.lax.dot_general(
            newpos_ref[...], w_ref[Q_DIM:, :].astype(jnp.bfloat16),
            (((1,), (0,)), ((), ())), preferred_element_type=jnp.float32)
        proj_out_ref[...] = proj + b_ref[...][None, :]

    dots = dotsb_ref[...]
    d = (qsq2_ref[...] - dots) + ksqb_ref[0:1, :]
    m = jnp.min(d, axis=1, keepdims=True)
    colf = lax.broadcasted_iota(jnp.int32, (1, PAIR), 1).astype(jnp.float32)
    argf = jnp.min(jnp.where(d == m, colf, _F32_BIG), axis=1, keepdims=True)
    arg = argf.astype(jnp.int32) + (i - 1) * PAIR
    better = m < best_ref[...]
    bidx_ref[...] = jnp.where(better, arg, bidx_ref[...])
    best_ref[...] = jnp.where(better, m, best_ref[...])

    tile_a = table_ref[:KT, :]
    dotsb_ref[:, :KT] = _mm_bf16(newpos_ref[...], tile_a)
    ksqb_ref[:, :KT] = jnp.broadcast_to(_ksq2(tile_a), (8, KT))
    tile_b = table_ref[KT:, :]
    dotsb_ref[:, KT:] = _mm_bf16(newpos_ref[...], tile_b)
    ksqb_ref[:, KT:] = jnp.broadcast_to(_ksq2(tile_b), (8, KT))

    @pl.when(i == GRID - 1)
    def _fin():
        idx_out_ref[...] = bidx_ref[...]


def _nn_and_project(actions, current_position, question_emb,
                    entity_table, W1_w, W1_b):
    return pl.pallas_call(
        _nn_body,
        grid=(GRID,),
        in_specs=[
            pl.BlockSpec((B, REL_DIM), lambda i: (0, 0)),
            pl.BlockSpec((B, ENT_DIM), lambda i: (0, 0)),
            pl.BlockSpec((B, Q_DIM), lambda i: (0, 0)),
            pl.BlockSpec((PAIR, ENT_DIM), lambda i: (jnp.minimum(i, NPAIR - 1), 0)),
            pl.BlockSpec((Q_DIM + ENT_DIM, HIST_DIM), lambda i: (0, 0)),
            pl.BlockSpec((HIST_DIM,), lambda i: (0,)),
        ],
        out_specs=[
            pl.BlockSpec((B, 1), lambda i: (0, 0)),
            pl.BlockSpec((B, HIST_DIM), lambda i: (0, 0)),
        ],
        out_shape=[
            jax.ShapeDtypeStruct((B, 1), jnp.int32),
            jax.ShapeDtypeStruct((B, HIST_DIM), jnp.float32),
        ],
        scratch_shapes=[
            pltpu.VMEM((B, ENT_DIM), jnp.bfloat16),
            pltpu.VMEM((B, 1), jnp.float32),
            pltpu.VMEM((B, 1), jnp.float32),
            pltpu.VMEM((B, 1), jnp.int32),
            pltpu.VMEM((B, PAIR), jnp.float32),
            pltpu.VMEM((8, PAIR), jnp.float32),
        ],
    )(actions, current_position, question_emb, entity_table, W1_w, W1_b)


@functools.cache
def _make_sc_gather():
    info = plsc.get_sparse_core_info()
    nc, ns = info.num_cores, info.num_subcores
    nw = nc * ns
    bpw = B // nw

    @functools.partial(
        pl.kernel,
        mesh=plsc.VectorSubcoreMesh(core_axis_name="c", subcore_axis_name="s"),
        out_type=jax.ShapeDtypeStruct((B, ENT_DIM), jnp.float32),
        scratch_types=[
            pltpu.VMEM((bpw,), jnp.int32),
            pltpu.VMEM((bpw, ENT_DIM), jnp.float32),
            pltpu.SemaphoreType.DMA,
        ],
    )
    def _sc_gather(table_hbm, idx_hbm, out_hbm, idx_v, rows_v, sem):
        wid = lax.axis_index("s") * nc + lax.axis_index("c")
        base = wid * bpw
        pltpu.sync_copy(idx_hbm.at[pl.ds(base, bpw)], idx_v)
        pltpu.async_copy(table_hbm.at[idx_v], rows_v, sem).wait()
        pltpu.sync_copy(rows_v, out_hbm.at[pl.ds(base, bpw)])

    return _sc_gather


def kernel(actions, current_position, question_emb, entity_table, W1_w, W1_b):
    idx, projected = _nn_and_project(
        actions, current_position, question_emb, entity_table, W1_w, W1_b)
    matched = _make_sc_gather()(entity_table, idx[:, 0])
    return matched, idx, projected


# speedup vs baseline: 1.3102x; 1.3102x over previous
import functools

import jax
import jax.numpy as jnp
from jax import lax
from jax.experimental import pallas as pl
from jax.experimental.pallas import tpu as pltpu
from jax.experimental.pallas import tpu_sc as plsc

B = 1024
ENT_DIM = 256
REL_DIM = 128
Q_DIM = 768
HIST_DIM = 768
K_ENT = 100000

KT = 2000
PAIR = 2 * KT
NPAIR = K_ENT // PAIR
GRID = NPAIR + 1

_F32_BIG = 3.0e38


def _mm_bf16(a, b_f32):
    return jax.lax.dot_general(
        a, b_f32.astype(jnp.bfloat16), (((1,), (1,)), ((), ())),
        preferred_element_type=jnp.float32)


def _ksq2(tile):
    s = 0.5 * jnp.sum(tile * tile, axis=1)
    return s[None, :]


def _nn_body(actions_ref, pos_ref, q_ref, table_ref, w_ref, b_ref,
             idx_out_ref, proj_out_ref,
             newpos_ref, qsq2_ref, best_ref, bidx_ref, dotsb_ref, ksqb_ref):
    i = pl.program_id(0)

    @pl.when(i == 0)
    def _init():
        phase = actions_ref[...]
        cos_p = jnp.cos(phase)
        sin_p = jnp.sin(phase)
        pos = pos_ref[...]
        re = pos[:, :REL_DIM]
        im = pos[:, REL_DIM:]
        re2 = re * cos_p - im * sin_p
        im2 = re * sin_p + im * cos_p
        new_pos = jnp.concatenate([re2, im2], axis=1)
        newpos_ref[...] = new_pos.astype(jnp.bfloat16)
        qsq2_ref[...] = 0.5 * jnp.sum(new_pos * new_pos, axis=1, keepdims=True)
        best_ref[...] = jnp.full((B, 1), jnp.inf, jnp.float32)
        bidx_ref[...] = jnp.zeros((B, 1), jnp.int32)
        dotsb_ref[...] = jnp.full((B, KT), -_F32_BIG, jnp.float32)
        ksqb_ref[...] = jnp.zeros((8, KT), jnp.float32)
        proj = jax.lax.dot_general(
            q_ref[...].astype(jnp.bfloat16), w_ref[:Q_DIM, :].astype(jnp.bfloat16),
            (((1,), (0,)), ((), ())), preferred_element_type=jnp.float32)
        proj += jax.lax.dot_general(
            newpos_ref[...], w_ref[Q_DIM:, :].astype(jnp.bfloat16),
            (((1,), (0,)), ((), ())), preferred_element_type=jnp.float32)
        proj_out_ref[...] = proj + b_ref[...][None, :]

    def epilogue(dots, ksq2, base):
        d = (qsq2_ref[...] - dots) + ksq2
        m = jnp.min(d, axis=1, keepdims=True)
        colf = lax.broadcasted_iota(jnp.int32, (1, KT), 1).astype(jnp.float32)
        argf = jnp.min(jnp.where(d == m, colf, _F32_BIG),
                       axis=1, keepdims=True)
        arg = argf.astype(jnp.int32) + base
        better = m < best_ref[...]
        bidx_ref[...] = jnp.where(better, arg, bidx_ref[...])
        best_ref[...] = jnp.where(better, m, best_ref[...])

    epilogue(dotsb_ref[...], ksqb_ref[0:1, :], i * PAIR - KT)

    tile_a = table_ref[:KT, :]
    epilogue(_mm_bf16(newpos_ref[...], tile_a), _ksq2(tile_a), i * PAIR)

    tile_b = table_ref[KT:, :]
    dotsb_ref[...] = _mm_bf16(newpos_ref[...], tile_b)
    ksqb_ref[...] = jnp.broadcast_to(_ksq2(tile_b), (8, KT))

    @pl.when(i == GRID - 1)
    def _fin():
        idx_out_ref[...] = bidx_ref[...]


def _nn_and_project(actions, current_position, question_emb,
                    entity_table, W1_w, W1_b):
    return pl.pallas_call(
        _nn_body,
        grid=(GRID,),
        in_specs=[
            pl.BlockSpec((B, REL_DIM), lambda i: (0, 0)),
            pl.BlockSpec((B, ENT_DIM), lambda i: (0, 0)),
            pl.BlockSpec((B, Q_DIM), lambda i: (0, 0)),
            pl.BlockSpec((PAIR, ENT_DIM), lambda i: (jnp.minimum(i, NPAIR - 1), 0)),
            pl.BlockSpec((Q_DIM + ENT_DIM, HIST_DIM), lambda i: (0, 0)),
            pl.BlockSpec((HIST_DIM,), lambda i: (0,)),
        ],
        out_specs=[
            pl.BlockSpec((B, 1), lambda i: (0, 0)),
            pl.BlockSpec((B, HIST_DIM), lambda i: (0, 0)),
        ],
        out_shape=[
            jax.ShapeDtypeStruct((B, 1), jnp.int32),
            jax.ShapeDtypeStruct((B, HIST_DIM), jnp.float32),
        ],
        scratch_shapes=[
            pltpu.VMEM((B, ENT_DIM), jnp.bfloat16),
            pltpu.VMEM((B, 1), jnp.float32),
            pltpu.VMEM((B, 1), jnp.float32),
            pltpu.VMEM((B, 1), jnp.int32),
            pltpu.VMEM((B, KT), jnp.float32),
            pltpu.VMEM((8, KT), jnp.float32),
        ],
    )(actions, current_position, question_emb, entity_table, W1_w, W1_b)


@functools.cache
def _make_sc_gather():
    info = plsc.get_sparse_core_info()
    nc, ns = info.num_cores, info.num_subcores
    nw = nc * ns
    bpw = B // nw

    @functools.partial(
        pl.kernel,
        mesh=plsc.VectorSubcoreMesh(core_axis_name="c", subcore_axis_name="s"),
        out_type=jax.ShapeDtypeStruct((B, ENT_DIM), jnp.float32),
        scratch_types=[
            pltpu.VMEM((bpw,), jnp.int32),
            pltpu.VMEM((bpw, ENT_DIM), jnp.float32),
            pltpu.SemaphoreType.DMA,
        ],
    )
    def _sc_gather(table_hbm, idx_hbm, out_hbm, idx_v, rows_v, sem):
        wid = lax.axis_index("s") * nc + lax.axis_index("c")
        base = wid * bpw
        pltpu.sync_copy(idx_hbm.at[pl.ds(base, bpw)], idx_v)
        pltpu.async_copy(table_hbm.at[idx_v], rows_v, sem).wait()
        pltpu.sync_copy(rows_v, out_hbm.at[pl.ds(base, bpw)])

    return _sc_gather


def kernel(actions, current_position, question_emb, entity_table, W1_w, W1_b):
    idx, projected = _nn_and_project(
        actions, current_position, question_emb, entity_table, W1_w, W1_b)
    matched = _make_sc_gather()(entity_table, idx[:, 0])
    return matched, idx, projected
